# baseline (speedup 1.0000x reference)
import functools

import jax
import jax.numpy as jnp
from jax import lax
from jax.experimental import pallas as pl
from jax.experimental.pallas import tpu as pltpu
from jax.experimental.pallas import tpu_sc as plsc

_NC = 2
_NS = 16
_LANES = 16


def _build_sc_kernel(B, L, Lc, D):
    NW = _NC * _NS
    WPB = NW // B
    TW = L // WPB
    C = 32
    NCH = TW // C
    NV = TW // _LANES
    ND = D // _LANES

    mesh = plsc.VectorSubcoreMesh(core_axis_name="c", subcore_axis_name="s")

    @functools.partial(
        pl.kernel,
        out_type=jax.ShapeDtypeStruct((B * L, D), jnp.float32),
        mesh=mesh,
        scratch_types=[
            pltpu.VMEM((L,), jnp.int32),
            pltpu.VMEM((TW,), jnp.int32),
            pltpu.VMEM((TW,), jnp.float32),
            pltpu.VMEM((_LANES,), jnp.int32),
            pltpu.VMEM((C + 8, D), jnp.float32),
            pltpu.VMEM((C, D), jnp.float32),
            pltpu.SemaphoreType.DMA,
        ],
        compiler_params=pltpu.CompilerParams(needs_layout_passes=False),
    )
    def dechunk(z_hbm, p_hbm, b_hbm, out_hbm,
                b_v, idx_v, p_v, pidx_v, rows_v, out_v, sem):
        wid = lax.axis_index("s") * _NC + lax.axis_index("c")
        batch = wid // WPB
        slot = wid % WPB
        t0 = slot * TW
        zbase = batch * Lc
        obase = batch * L + t0

        pltpu.sync_copy(b_hbm.at[pl.ds(batch * L, L)], b_v)
        pltpu.sync_copy(p_hbm.at[pl.ds(obase, TW)], p_v)

        def pref_body(j, carry):
            bv = b_v[pl.ds(j * _LANES, _LANES)]
            bb = jnp.where(bv == 1, 1, 0).astype(jnp.int32)
            return carry + jnp.sum(bb)

        carry0 = lax.fori_loop(0, slot * NV, pref_body, jnp.int32(0))

        def scan_body(j, carry):
            bv = b_v[pl.ds(t0 + j * _LANES, _LANES)]
            bb = jnp.where(bv == 1, 1, 0).astype(jnp.int32)
            incl = plsc.cumsum(bb)
            excl = carry + (incl - bb)
            idx_v[pl.ds(j * _LANES, _LANES)] = jnp.minimum(excl, Lc - 1) + zbase
            return carry + jnp.sum(bb)

        lax.fori_loop(0, NV, scan_body, carry0)

        bv_last = b_v[pl.ds(jnp.maximum(t0 - _LANES, 0), _LANES)]
        bb_last = jnp.where(bv_last[_LANES - 1] == 1, 1, 0).astype(jnp.int32)
        prev_idx = jnp.where(
            t0 > 0, jnp.minimum(carry0 - bb_last, Lc - 1), 0) + zbase
        pidx_v[pl.ds(0, _LANES)] = jnp.full((_LANES,), prev_idx, jnp.int32)
        pltpu.async_copy(
            z_hbm.at[pidx_v.at[pl.ds(0, 8)]], rows_v.at[pl.ds(0, 8)], sem
        ).wait()

        def chunk_body(ci, _):
            s = ci * C
            pltpu.async_copy(
                z_hbm.at[idx_v.at[pl.ds(s, C)]], rows_v.at[pl.ds(8, C)], sem
            ).wait()

            def row_body(i, _2):
                pv = plsc.load_gather(
                    p_v, [jnp.full((_LANES,), s + i, jnp.int32)])

                def d_body(dj, _3):
                    prev = rows_v[i + 7, pl.ds(dj * _LANES, _LANES)]
                    cur = rows_v[i + 8, pl.ds(dj * _LANES, _LANES)]
                    out_v[i, pl.ds(dj * _LANES, _LANES)] = prev + pv * (cur - prev)
                    return 0

                lax.fori_loop(0, ND, d_body, 0)
                return 0

            lax.fori_loop(0, C, row_body, 0)

            def cp_body(dj, _2):
                rows_v[7, pl.ds(dj * _LANES, _LANES)] = (
                    rows_v[C + 7, pl.ds(dj * _LANES, _LANES)])
                return 0

            lax.fori_loop(0, ND, cp_body, 0)

            pltpu.sync_copy(out_v, out_hbm.at[pl.ds(obase + s, C)])
            return 0

        lax.fori_loop(0, NCH, chunk_body, 0)

    return dechunk


def kernel(z, p, b, original_len):
    B, Lc, D = z.shape
    L = p.shape[1]
    z2d = z.reshape(B * Lc, D)
    p1 = p.reshape(B * L)
    b1 = b.reshape(B * L).astype(jnp.int32)
    out = _build_sc_kernel(B, L, Lc, D)(z2d, p1, b1)
    return out.reshape(B, L, D)


# speedup vs baseline: 4.3740x; 4.3740x over previous
import functools

import jax
import jax.numpy as jnp
from jax import lax
from jax.experimental import pallas as pl
from jax.experimental.pallas import tpu as pltpu
from jax.experimental.pallas import tpu_sc as plsc

_NC = 2
_NS = 16
_LANES = 16


def _build_sc_kernel(B, L, Lc, D):
    NW = _NC * _NS
    WPB = NW // B
    TW = L // WPB
    C = 32
    NCH = TW // C
    NV = TW // _LANES
    ND = D // _LANES
    UNR = 8

    mesh = plsc.VectorSubcoreMesh(core_axis_name="c", subcore_axis_name="s")

    @functools.partial(
        pl.kernel,
        out_type=jax.ShapeDtypeStruct((B * L, D), jnp.float32),
        mesh=mesh,
        scratch_types=[
            pltpu.VMEM((L,), jnp.int32),
            pltpu.VMEM((TW,), jnp.int32),
            pltpu.VMEM((TW,), jnp.float32),
            pltpu.VMEM((_LANES,), jnp.int32),
            pltpu.VMEM((C + 8, D), jnp.float32),
            pltpu.VMEM((C + 8, D), jnp.float32),
            pltpu.SMEM((TW,), jnp.float32),
            pltpu.SemaphoreType.DMA,
            pltpu.SemaphoreType.DMA,
        ],
        compiler_params=pltpu.CompilerParams(needs_layout_passes=False),
    )
    def dechunk(z_hbm, p_hbm, b_hbm, out_hbm,
                b_v, idx_v, p_v, pidx_v, rows0, rows1, p_s, gsem, osem):
        wid = lax.axis_index("s") * _NC + lax.axis_index("c")
        batch = wid // WPB
        slot = wid % WPB
        t0 = slot * TW
        zbase = batch * Lc
        obase = batch * L + t0

        pltpu.sync_copy(b_hbm.at[pl.ds(batch * L, L)], b_v)
        pltpu.sync_copy(p_hbm.at[pl.ds(obase, TW)], p_v)

        def pfill_body(j, _):
            v = p_v[pl.ds(j * _LANES, _LANES)]
            for lane in range(_LANES):
                p_s[j * _LANES + lane] = v[lane]
            return 0

        lax.fori_loop(0, NV, pfill_body, 0)

        def pref_body(j, carry):
            bv = b_v[pl.ds(j * _LANES, _LANES)]
            bb = jnp.where(bv == 1, 1, 0).astype(jnp.int32)
            return carry + jnp.sum(bb)

        carry0 = lax.fori_loop(0, slot * NV, pref_body, jnp.int32(0))

        def scan_body(j, carry):
            bv = b_v[pl.ds(t0 + j * _LANES, _LANES)]
            bb = jnp.where(bv == 1, 1, 0).astype(jnp.int32)
            incl = plsc.cumsum(bb)
            excl = carry + (incl - bb)
            idx_v[pl.ds(j * _LANES, _LANES)] = jnp.minimum(excl, Lc - 1) + zbase
            return carry + jnp.sum(bb)

        lax.fori_loop(0, NV, scan_body, carry0)

        bv_last = b_v[pl.ds(jnp.maximum(t0 - _LANES, 0), _LANES)]
        bb_last = jnp.where(bv_last[_LANES - 1] == 1, 1, 0).astype(jnp.int32)
        prev_idx = jnp.where(
            t0 > 0, jnp.minimum(carry0 - bb_last, Lc - 1), 0) + zbase
        pidx_v[pl.ds(0, _LANES)] = jnp.full((_LANES,), prev_idx, jnp.int32)
        pltpu.async_copy(
            z_hbm.at[pidx_v.at[pl.ds(0, 8)]], rows0.at[pl.ds(0, 8)], gsem
        ).wait()

        pltpu.async_copy(
            z_hbm.at[idx_v.at[pl.ds(0, C)]], rows0.at[pl.ds(8, C)], gsem)

        def copy_last(src, dst):
            def cp_body(dj, _):
                dst[7, pl.ds(dj * _LANES, _LANES)] = (
                    src[C + 7, pl.ds(dj * _LANES, _LANES)])
                return 0

            lax.fori_loop(0, ND, cp_body, 0)

        def blend(buf, s):
            def d_body(dj, _):
                col = dj * _LANES
                prev0 = buf[7, pl.ds(col, _LANES)]

                def i_body(u, prev):
                    for step in range(UNR):
                        i = u * UNR + step
                        pv = jnp.full((_LANES,), p_s[s + i], jnp.float32)
                        cur = buf[8 + i, pl.ds(col, _LANES)]
                        buf[8 + i, pl.ds(col, _LANES)] = prev + pv * (cur - prev)
                        prev = cur
                    return prev

                lax.fori_loop(0, C // UNR, i_body, prev0)
                return 0

            lax.fori_loop(0, ND, d_body, 0)

        def phase(k, buf_a, buf_b):
            s = k * C
            pltpu.make_async_copy(
                z_hbm.at[pl.ds(0, C)], buf_a.at[pl.ds(8, C)], gsem).wait()
            copy_last(buf_a, buf_b)

            @pl.when(k >= 1)
            def _():
                pltpu.make_async_copy(
                    buf_b.at[pl.ds(8, C)], out_hbm.at[pl.ds(0, C)], osem
                ).wait()

            @pl.when(k + 1 < NCH)
            def _():
                pltpu.async_copy(
                    z_hbm.at[idx_v.at[pl.ds((k + 1) * C, C)]],
                    buf_b.at[pl.ds(8, C)], gsem)

            blend(buf_a, s)
            pltpu.async_copy(
                buf_a.at[pl.ds(8, C)], out_hbm.at[pl.ds(obase + s, C)], osem)

        def pair_body(g, _):
            phase(2 * g, rows0, rows1)
            phase(2 * g + 1, rows1, rows0)
            return 0

        lax.fori_loop(0, NCH // 2, pair_body, 0)
        pltpu.make_async_copy(
            rows1.at[pl.ds(8, C)], out_hbm.at[pl.ds(0, C)], osem).wait()

    return dechunk


def kernel(z, p, b, original_len):
    B, Lc, D = z.shape
    L = p.shape[1]
    z2d = z.reshape(B * Lc, D)
    p1 = p.reshape(B * L)
    b1 = b.reshape(B * L).astype(jnp.int32)
    out = _build_sc_kernel(B, L, Lc, D)(z2d, p1, b1)
    return out.reshape(B, L, D)
